# baseline (speedup 1.0000x reference)
import functools

import jax
import jax.numpy as jnp
from jax import lax
from jax.experimental import pallas as pl
from jax.experimental.pallas import tpu as pltpu
from jax.experimental.pallas import tpu_sc as plsc

N = 10000
E = 320000
D = 128
B = 64

NC = 2
NS = 16
NW = NC * NS
CHUNK = 128
NCHUNK = 80
EPAD = NW * NCHUNK * CHUNK
NPAD = 10240
ZROWS = NPAD // NS
PAD_DST = N


def _sc_scatter_body(x_hbm, src_hbm, dst_hbm, zeros_hbm, out_hbm,
                     src_v, dst_v, gbuf, agg, gsem):
    c = lax.axis_index("c")
    s = lax.axis_index("s")
    wid = c * NS + s

    pltpu.sync_copy(zeros_hbm, agg.at[pl.ds(s * ZROWS, ZROWS)])
    plsc.subcore_barrier()

    pltpu.sync_copy(src_hbm.at[wid], src_v)
    pltpu.sync_copy(dst_hbm.at[wid], dst_v)

    def body(j, carry):
        pltpu.async_copy(x_hbm.at[src_v.at[j]], gbuf, gsem).wait()
        pltpu.sync_copy(gbuf, agg.at[dst_v.at[j]], add=True)
        return carry

    lax.fori_loop(0, NCHUNK, body, 0)
    plsc.subcore_barrier()

    pltpu.sync_copy(agg.at[pl.ds(s * ZROWS, ZROWS)],
                    out_hbm.at[pl.ds(c * NPAD + s * ZROWS, ZROWS)])


_sc_scatter = pl.kernel(
    _sc_scatter_body,
    out_type=jax.ShapeDtypeStruct((NC * NPAD, D), jnp.float32),
    mesh=plsc.VectorSubcoreMesh(core_axis_name="c", subcore_axis_name="s"),
    scratch_types=[
        pltpu.VMEM((NCHUNK, CHUNK), jnp.int32),
        pltpu.VMEM((NCHUNK, CHUNK), jnp.int32),
        pltpu.VMEM((CHUNK, D), jnp.float32),
        pltpu.VMEM_SHARED((NPAD, D), jnp.float32),
        pltpu.SemaphoreType.DMA,
    ],
)


def _dense_body(x_ref, a0_ref, a1_ref, Wa_ref, ba_ref, g_ref, bt_ref,
                Wb_ref, bb_ref, out_ref):
    h = x_ref[...] + a0_ref[...] + a1_ref[...]
    h1 = jnp.dot(h, Wa_ref[...], preferred_element_type=jnp.float32,
                 precision=lax.Precision.HIGHEST) + ba_ref[...]
    m = jnp.mean(h1, axis=0, keepdims=True)
    v = jnp.mean((h1 - m) ** 2, axis=0, keepdims=True)
    hn = (h1 - m) * lax.rsqrt(v + 1e-5) * g_ref[...] + bt_ref[...]
    hn = jnp.maximum(hn, 0.0)
    h2 = jnp.dot(hn, Wb_ref[...], preferred_element_type=jnp.float32,
                 precision=lax.Precision.HIGHEST) + bb_ref[...]
    out_ref[...] = jnp.maximum(h2, 0.0)


_dense = pl.pallas_call(
    _dense_body,
    out_shape=jax.ShapeDtypeStruct((N, D), jnp.float32),
)


def _dense_pool_body(x_ref, a0_ref, a1_ref, Wa_ref, ba_ref, g_ref, bt_ref,
                     Wb_ref, bb_ref, starts_ref, ends_ref, out_ref):
    h = x_ref[...] + a0_ref[...] + a1_ref[...]
    h1 = jnp.dot(h, Wa_ref[...], preferred_element_type=jnp.float32,
                 precision=lax.Precision.HIGHEST) + ba_ref[...]
    m = jnp.mean(h1, axis=0, keepdims=True)
    v = jnp.mean((h1 - m) ** 2, axis=0, keepdims=True)
    hn = (h1 - m) * lax.rsqrt(v + 1e-5) * g_ref[...] + bt_ref[...]
    hn = jnp.maximum(hn, 0.0)
    h2 = jnp.dot(hn, Wb_ref[...], preferred_element_type=jnp.float32,
                 precision=lax.Precision.HIGHEST) + bb_ref[...]
    h2 = jnp.maximum(h2, 0.0)
    rows = lax.broadcasted_iota(jnp.int32, (B, N), 1)
    mask = (rows >= starts_ref[...]) & (rows < ends_ref[...])
    out_ref[...] = jnp.dot(mask.astype(jnp.float32), h2,
                           preferred_element_type=jnp.float32,
                           precision=lax.Precision.HIGHEST)


_dense_pool = pl.pallas_call(
    _dense_pool_body,
    out_shape=jax.ShapeDtypeStruct((B, D), jnp.float32),
)


def kernel(x, edge_index, batch,
           W1_1, b1_1, gamma1, beta1, W1_2, b1_2,
           W4_1, b4_1, gamma4, beta4, W4_2, b4_2,
           W5_1, b5_1, gamma5, beta5, W5_2, b5_2):
    npad_e = EPAD - E
    src = jnp.concatenate(
        [edge_index[0].astype(jnp.int32),
         jnp.zeros((npad_e,), jnp.int32)]).reshape(NW, NCHUNK, CHUNK)
    pad_rows = PAD_DST + jnp.arange(npad_e, dtype=jnp.int32) % (NPAD - N)
    dst = jnp.concatenate(
        [edge_index[1].astype(jnp.int32),
         pad_rows]).reshape(NW, NCHUNK, CHUNK)
    zeros = jnp.zeros((ZROWS, D), jnp.float32)
    batch32 = batch.astype(jnp.int32)
    starts = batch32[:-1].reshape(B, 1)
    ends = batch32[1:].reshape(B, 1)

    def layer(h, Wa, ba, g, bt, Wb, bb, last):
        agg = _sc_scatter(h, src, dst, zeros)
        a0 = agg[:N]
        a1 = agg[NPAD:NPAD + N]
        ba2 = ba.reshape(1, D)
        g2 = g.reshape(1, D)
        bt2 = bt.reshape(1, D)
        bb2 = bb.reshape(1, D)
        if last:
            return _dense_pool(h, a0, a1, Wa, ba2, g2, bt2, Wb, bb2,
                               starts, ends)
        return _dense(h, a0, a1, Wa, ba2, g2, bt2, Wb, bb2)

    h = layer(x, W1_1, b1_1, gamma1, beta1, W1_2, b1_2, False)
    h = layer(h, W4_1, b4_1, gamma4, beta4, W4_2, b4_2, False)
    out = layer(h, W5_1, b5_1, gamma5, beta5, W5_2, b5_2, True)
    return out


# speedup vs baseline: 2.5058x; 2.5058x over previous
import functools

import jax
import jax.numpy as jnp
from jax import lax
from jax.experimental import pallas as pl
from jax.experimental.pallas import tpu as pltpu
from jax.experimental.pallas import tpu_sc as plsc

N = 10000
E = 320000
D = 128
B = 64

NC = 2
NS = 16
NW = NC * NS
CHUNK = 128
NCHUNK = 80
EPAD = NW * NCHUNK * CHUNK
NPAD = 10240
ZROWS = NPAD // NS
PAD_DST = N


def _sc_scatter_body(x_hbm, src_hbm, dst_hbm, zeros_hbm, out_hbm,
                     src_v, dst_v, gbuf, agg, gsem):
    c = lax.axis_index("c")
    s = lax.axis_index("s")
    wid = c * NS + s

    pltpu.sync_copy(zeros_hbm, agg.at[pl.ds(s * ZROWS, ZROWS)])
    plsc.subcore_barrier()

    pltpu.sync_copy(src_hbm.at[wid], src_v)
    pltpu.sync_copy(dst_hbm.at[wid], dst_v)

    def body(j, carry):
        pltpu.async_copy(x_hbm.at[src_v.at[j]], gbuf, gsem).wait()
        pltpu.sync_copy(gbuf, agg.at[dst_v.at[j]], add=True)
        return carry

    lax.fori_loop(0, NCHUNK, body, 0)
    plsc.subcore_barrier()

    pltpu.sync_copy(agg.at[pl.ds(s * ZROWS, ZROWS)],
                    out_hbm.at[pl.ds(c * NPAD + s * ZROWS, ZROWS)])


_sc_scatter = pl.kernel(
    _sc_scatter_body,
    out_type=jax.ShapeDtypeStruct((NC * NPAD, D), jnp.float32),
    mesh=plsc.VectorSubcoreMesh(core_axis_name="c", subcore_axis_name="s"),
    scratch_types=[
        pltpu.VMEM((NCHUNK, CHUNK), jnp.int32),
        pltpu.VMEM((NCHUNK, CHUNK), jnp.int32),
        pltpu.VMEM((CHUNK, D), jnp.float32),
        pltpu.VMEM_SHARED((NPAD, D), jnp.float32),
        pltpu.SemaphoreType.DMA,
    ],
)


def _dense_body(x_ref, a0_ref, a1_ref, Wa_ref, ba_ref, g_ref, bt_ref,
                Wb_ref, bb_ref, out_ref):
    h = x_ref[...] + a0_ref[...] + a1_ref[...]
    h1 = jnp.dot(h, Wa_ref[...], preferred_element_type=jnp.float32,
                 precision=lax.Precision.HIGHEST) + ba_ref[...]
    m = jnp.mean(h1, axis=0, keepdims=True)
    v = jnp.mean((h1 - m) ** 2, axis=0, keepdims=True)
    hn = (h1 - m) * lax.rsqrt(v + 1e-5) * g_ref[...] + bt_ref[...]
    hn = jnp.maximum(hn, 0.0)
    h2 = jnp.dot(hn, Wb_ref[...], preferred_element_type=jnp.float32,
                 precision=lax.Precision.HIGHEST) + bb_ref[...]
    out_ref[...] = jnp.maximum(h2, 0.0)


_dense = pl.pallas_call(
    _dense_body,
    out_shape=jax.ShapeDtypeStruct((N, D), jnp.float32),
)


def _dense_pool_body(x_ref, a0_ref, a1_ref, Wa_ref, ba_ref, g_ref, bt_ref,
                     Wb_ref, bb_ref, starts_ref, ends_ref, out_ref):
    h = x_ref[...] + a0_ref[...] + a1_ref[...]
    h1 = jnp.dot(h, Wa_ref[...], preferred_element_type=jnp.float32,
                 precision=lax.Precision.HIGHEST) + ba_ref[...]
    m = jnp.mean(h1, axis=0, keepdims=True)
    v = jnp.mean((h1 - m) ** 2, axis=0, keepdims=True)
    hn = (h1 - m) * lax.rsqrt(v + 1e-5) * g_ref[...] + bt_ref[...]
    hn = jnp.maximum(hn, 0.0)
    h2 = jnp.dot(hn, Wb_ref[...], preferred_element_type=jnp.float32,
                 precision=lax.Precision.HIGHEST) + bb_ref[...]
    h2 = jnp.maximum(h2, 0.0)
    rows = lax.broadcasted_iota(jnp.int32, (B, N), 1)
    mask = (rows >= starts_ref[...]) & (rows < ends_ref[...])
    out_ref[...] = jnp.dot(mask.astype(jnp.float32), h2,
                           preferred_element_type=jnp.float32,
                           precision=lax.Precision.HIGHEST)


_dense_pool = pl.pallas_call(
    _dense_pool_body,
    out_shape=jax.ShapeDtypeStruct((B, D), jnp.float32),
)


def kernel(x, edge_index, batch,
           W1_1, b1_1, gamma1, beta1, W1_2, b1_2,
           W4_1, b4_1, gamma4, beta4, W4_2, b4_2,
           W5_1, b5_1, gamma5, beta5, W5_2, b5_2):
    ppt = NCHUNK * CHUNK - E // NW
    pad_src = (jnp.arange(ppt, dtype=jnp.int32) * 41) % N
    pad_dst = PAD_DST + jnp.arange(ppt, dtype=jnp.int32)
    src = jnp.concatenate(
        [edge_index[0].astype(jnp.int32).reshape(NW, E // NW),
         jnp.broadcast_to(pad_src, (NW, ppt))], axis=1
    ).reshape(NW, NCHUNK, CHUNK)
    dst = jnp.concatenate(
        [edge_index[1].astype(jnp.int32).reshape(NW, E // NW),
         jnp.broadcast_to(pad_dst, (NW, ppt))], axis=1
    ).reshape(NW, NCHUNK, CHUNK)
    zeros = jnp.zeros((ZROWS, D), jnp.float32)
    batch32 = batch.astype(jnp.int32)
    starts = batch32[:-1].reshape(B, 1)
    ends = batch32[1:].reshape(B, 1)

    def layer(h, Wa, ba, g, bt, Wb, bb, last):
        agg = _sc_scatter(h, src, dst, zeros)
        a0 = agg[:N]
        a1 = agg[NPAD:NPAD + N]
        ba2 = ba.reshape(1, D)
        g2 = g.reshape(1, D)
        bt2 = bt.reshape(1, D)
        bb2 = bb.reshape(1, D)
        if last:
            return _dense_pool(h, a0, a1, Wa, ba2, g2, bt2, Wb, bb2,
                               starts, ends)
        return _dense(h, a0, a1, Wa, ba2, g2, bt2, Wb, bb2)

    h = layer(x, W1_1, b1_1, gamma1, beta1, W1_2, b1_2, False)
    h = layer(h, W4_1, b4_1, gamma4, beta4, W4_2, b4_2, False)
    out = layer(h, W5_1, b5_1, gamma5, beta5, W5_2, b5_2, True)
    return out


# speedup vs baseline: 3.5402x; 1.4128x over previous
import functools

import jax
import jax.numpy as jnp
from jax import lax
from jax.experimental import pallas as pl
from jax.experimental.pallas import tpu as pltpu
from jax.experimental.pallas import tpu_sc as plsc

N = 10000
E = 320000
D = 128
B = 64

NC = 2
NS = 16
NW = NC * NS
CHUNK = 128
NCHUNK = 80
EPAD = NW * NCHUNK * CHUNK
NPAD = 10240
ZROWS = NPAD // NS
PAD_DST = N


PCH = NCHUNK // 2
NBUF = 2


def _sc_scatter_body(x_hbm, src_hbm, dst_hbm, zeros_hbm, out_hbm,
                     src_v, dst_v, gbuf0, gbuf1, agg, sem0, sem1):
    c = lax.axis_index("c")
    s = lax.axis_index("s")
    wid = c * NS + s
    gbufs = (gbuf0, gbuf1)
    gsems = (sem0, sem1)

    def wait_gather(b):
        pltpu.make_async_copy(x_hbm.at[pl.ds(0, CHUNK)], gbufs[b],
                              gsems[b]).wait()

    pltpu.sync_copy(zeros_hbm, agg.at[pl.ds(s * ZROWS, ZROWS)])
    plsc.subcore_barrier()

    for p in range(NCHUNK // PCH):
        pltpu.sync_copy(src_hbm.at[wid, pl.ds(p * PCH, PCH)], src_v)
        pltpu.sync_copy(dst_hbm.at[wid, pl.ds(p * PCH, PCH)], dst_v)
        for b in range(NBUF):
            pltpu.async_copy(x_hbm.at[src_v.at[b]], gbufs[b], gsems[b])

        def body(jj, carry):
            for b in range(NBUF):
                j = jj * NBUF + b
                wait_gather(b)
                pltpu.sync_copy(gbufs[b], agg.at[dst_v.at[j]], add=True)
                pltpu.async_copy(x_hbm.at[src_v.at[j + NBUF]], gbufs[b],
                                 gsems[b])
            return carry

        lax.fori_loop(0, PCH // NBUF - 1, body, 0)
        for b in range(NBUF):
            j = PCH - NBUF + b
            wait_gather(b)
            pltpu.sync_copy(gbufs[b], agg.at[dst_v.at[j]], add=True)

    plsc.subcore_barrier()

    pltpu.sync_copy(agg.at[pl.ds(s * ZROWS, ZROWS)],
                    out_hbm.at[pl.ds(c * NPAD + s * ZROWS, ZROWS)])


_sc_scatter = pl.kernel(
    _sc_scatter_body,
    out_type=jax.ShapeDtypeStruct((NC * NPAD, D), jnp.float32),
    mesh=plsc.VectorSubcoreMesh(core_axis_name="c", subcore_axis_name="s"),
    scratch_types=[
        pltpu.VMEM((PCH, CHUNK), jnp.int32),
        pltpu.VMEM((PCH, CHUNK), jnp.int32),
        pltpu.VMEM((CHUNK, D), jnp.float32),
        pltpu.VMEM((CHUNK, D), jnp.float32),
        pltpu.VMEM_SHARED((NPAD, D), jnp.float32),
        pltpu.SemaphoreType.DMA,
        pltpu.SemaphoreType.DMA,
    ],
)


def _dense_body(x_ref, a0_ref, a1_ref, Wa_ref, ba_ref, g_ref, bt_ref,
                Wb_ref, bb_ref, out_ref):
    h = x_ref[...] + a0_ref[...] + a1_ref[...]
    h1 = jnp.dot(h, Wa_ref[...], preferred_element_type=jnp.float32,
                 precision=lax.Precision.HIGHEST) + ba_ref[...]
    m = jnp.mean(h1, axis=0, keepdims=True)
    v = jnp.mean((h1 - m) ** 2, axis=0, keepdims=True)
    hn = (h1 - m) * lax.rsqrt(v + 1e-5) * g_ref[...] + bt_ref[...]
    hn = jnp.maximum(hn, 0.0)
    h2 = jnp.dot(hn, Wb_ref[...], preferred_element_type=jnp.float32,
                 precision=lax.Precision.HIGHEST) + bb_ref[...]
    out_ref[...] = jnp.maximum(h2, 0.0)


_dense = pl.pallas_call(
    _dense_body,
    out_shape=jax.ShapeDtypeStruct((N, D), jnp.float32),
)


def _dense_pool_body(x_ref, a0_ref, a1_ref, Wa_ref, ba_ref, g_ref, bt_ref,
                     Wb_ref, bb_ref, starts_ref, ends_ref, out_ref):
    h = x_ref[...] + a0_ref[...] + a1_ref[...]
    h1 = jnp.dot(h, Wa_ref[...], preferred_element_type=jnp.float32,
                 precision=lax.Precision.HIGHEST) + ba_ref[...]
    m = jnp.mean(h1, axis=0, keepdims=True)
    v = jnp.mean((h1 - m) ** 2, axis=0, keepdims=True)
    hn = (h1 - m) * lax.rsqrt(v + 1e-5) * g_ref[...] + bt_ref[...]
    hn = jnp.maximum(hn, 0.0)
    h2 = jnp.dot(hn, Wb_ref[...], preferred_element_type=jnp.float32,
                 precision=lax.Precision.HIGHEST) + bb_ref[...]
    h2 = jnp.maximum(h2, 0.0)
    rows = lax.broadcasted_iota(jnp.int32, (B, N), 1)
    mask = (rows >= starts_ref[...]) & (rows < ends_ref[...])
    out_ref[...] = jnp.dot(mask.astype(jnp.float32), h2,
                           preferred_element_type=jnp.float32,
                           precision=lax.Precision.HIGHEST)


_dense_pool = pl.pallas_call(
    _dense_pool_body,
    out_shape=jax.ShapeDtypeStruct((B, D), jnp.float32),
)


def kernel(x, edge_index, batch,
           W1_1, b1_1, gamma1, beta1, W1_2, b1_2,
           W4_1, b4_1, gamma4, beta4, W4_2, b4_2,
           W5_1, b5_1, gamma5, beta5, W5_2, b5_2):
    ppt = NCHUNK * CHUNK - E // NW
    pad_src = (jnp.arange(ppt, dtype=jnp.int32) * 41) % N
    pad_dst = PAD_DST + jnp.arange(ppt, dtype=jnp.int32)
    src = jnp.concatenate(
        [edge_index[0].astype(jnp.int32).reshape(NW, E // NW),
         jnp.broadcast_to(pad_src, (NW, ppt))], axis=1
    ).reshape(NW, NCHUNK, CHUNK)
    dst = jnp.concatenate(
        [edge_index[1].astype(jnp.int32).reshape(NW, E // NW),
         jnp.broadcast_to(pad_dst, (NW, ppt))], axis=1
    ).reshape(NW, NCHUNK, CHUNK)
    zeros = jnp.zeros((ZROWS, D), jnp.float32)
    batch32 = batch.astype(jnp.int32)
    starts = batch32[:-1].reshape(B, 1)
    ends = batch32[1:].reshape(B, 1)

    def layer(h, Wa, ba, g, bt, Wb, bb, last):
        agg = _sc_scatter(h, src, dst, zeros)
        a0 = agg[:N]
        a1 = agg[NPAD:NPAD + N]
        ba2 = ba.reshape(1, D)
        g2 = g.reshape(1, D)
        bt2 = bt.reshape(1, D)
        bb2 = bb.reshape(1, D)
        if last:
            return _dense_pool(h, a0, a1, Wa, ba2, g2, bt2, Wb, bb2,
                               starts, ends)
        return _dense(h, a0, a1, Wa, ba2, g2, bt2, Wb, bb2)

    h = layer(x, W1_1, b1_1, gamma1, beta1, W1_2, b1_2, False)
    h = layer(h, W4_1, b4_1, gamma4, beta4, W4_2, b4_2, False)
    out = layer(h, W5_1, b5_1, gamma5, beta5, W5_2, b5_2, True)
    return out


# speedup vs baseline: 4.1170x; 1.1629x over previous
import functools

import jax
import jax.numpy as jnp
from jax import lax
from jax.experimental import pallas as pl
from jax.experimental.pallas import tpu as pltpu
from jax.experimental.pallas import tpu_sc as plsc

N = 10000
E = 320000
D = 128
B = 64

NC = 2
NS = 16
NW = NC * NS
CHUNK = 128
NCHUNK = 80
EPAD = NW * NCHUNK * CHUNK
NPAD = 10240
ZROWS = NPAD // NS
PAD_DST = N


PCH = NCHUNK // 2
NBUF = 2


def _sc_scatter_body(x_hbm, src_hbm, dst_hbm, zeros_hbm, out_hbm,
                     src_v, dst_v, gbuf0, gbuf1, agg, sem0, sem1):
    c = lax.axis_index("c")
    s = lax.axis_index("s")
    wid = c * NS + s
    gbufs = (gbuf0, gbuf1)
    gsems = (sem0, sem1)

    def wait_gather(b):
        pltpu.make_async_copy(x_hbm.at[pl.ds(0, CHUNK)], gbufs[b],
                              gsems[b]).wait()

    pltpu.sync_copy(zeros_hbm, agg.at[pl.ds(s * ZROWS, ZROWS)])
    plsc.subcore_barrier()

    for p in range(NCHUNK // PCH):
        pltpu.sync_copy(src_hbm.at[wid, pl.ds(p * PCH, PCH)], src_v)
        pltpu.sync_copy(dst_hbm.at[wid, pl.ds(p * PCH, PCH)], dst_v)
        for b in range(NBUF):
            pltpu.async_copy(x_hbm.at[src_v.at[b]], gbufs[b], gsems[b])

        def body(jj, carry):
            for b in range(NBUF):
                j = jj * NBUF + b
                wait_gather(b)
                pltpu.sync_copy(gbufs[b], agg.at[dst_v.at[j]], add=True)
                pltpu.async_copy(x_hbm.at[src_v.at[j + NBUF]], gbufs[b],
                                 gsems[b])
            return carry

        lax.fori_loop(0, PCH // NBUF - 1, body, 0)
        for b in range(NBUF):
            j = PCH - NBUF + b
            wait_gather(b)
            pltpu.sync_copy(gbufs[b], agg.at[dst_v.at[j]], add=True)

    plsc.subcore_barrier()

    pltpu.sync_copy(agg.at[pl.ds(s * ZROWS, ZROWS)],
                    out_hbm.at[pl.ds(c * NPAD + s * ZROWS, ZROWS)])


_sc_scatter = pl.kernel(
    _sc_scatter_body,
    out_type=jax.ShapeDtypeStruct((NC * NPAD, D), jnp.float32),
    mesh=plsc.VectorSubcoreMesh(core_axis_name="c", subcore_axis_name="s"),
    scratch_types=[
        pltpu.VMEM((PCH, CHUNK), jnp.int32),
        pltpu.VMEM((PCH, CHUNK), jnp.int32),
        pltpu.VMEM((CHUNK, D), jnp.float32),
        pltpu.VMEM((CHUNK, D), jnp.float32),
        pltpu.VMEM_SHARED((NPAD, D), jnp.float32),
        pltpu.SemaphoreType.DMA,
        pltpu.SemaphoreType.DMA,
    ],
)


def _dense_body(x_ref, agg_ref, Wa_ref, ba_ref, g_ref, bt_ref,
                Wb_ref, bb_ref, out_ref):
    h = x_ref[...] + agg_ref[0:N] + agg_ref[NPAD:NPAD + N]
    h1 = jnp.dot(h, Wa_ref[...], preferred_element_type=jnp.float32) + ba_ref[...]
    m = jnp.mean(h1, axis=0, keepdims=True)
    v = jnp.mean((h1 - m) ** 2, axis=0, keepdims=True)
    hn = (h1 - m) * lax.rsqrt(v + 1e-5) * g_ref[...] + bt_ref[...]
    hn = jnp.maximum(hn, 0.0)
    h2 = jnp.dot(hn, Wb_ref[...], preferred_element_type=jnp.float32) + bb_ref[...]
    out_ref[...] = jnp.maximum(h2, 0.0)


_dense = pl.pallas_call(
    _dense_body,
    out_shape=jax.ShapeDtypeStruct((N, D), jnp.float32),
)


def _dense_pool_body(x_ref, agg_ref, Wa_ref, ba_ref, g_ref, bt_ref,
                     Wb_ref, bb_ref, starts_ref, ends_ref, out_ref):
    h = x_ref[...] + agg_ref[0:N] + agg_ref[NPAD:NPAD + N]
    h1 = jnp.dot(h, Wa_ref[...], preferred_element_type=jnp.float32) + ba_ref[...]
    m = jnp.mean(h1, axis=0, keepdims=True)
    v = jnp.mean((h1 - m) ** 2, axis=0, keepdims=True)
    hn = (h1 - m) * lax.rsqrt(v + 1e-5) * g_ref[...] + bt_ref[...]
    hn = jnp.maximum(hn, 0.0)
    h2 = jnp.dot(hn, Wb_ref[...], preferred_element_type=jnp.float32) + bb_ref[...]
    h2 = jnp.maximum(h2, 0.0)
    rows = lax.broadcasted_iota(jnp.int32, (B, N), 1)
    mask = (rows >= starts_ref[...]) & (rows < ends_ref[...])
    out_ref[...] = jnp.dot(mask.astype(jnp.float32), h2,
                           preferred_element_type=jnp.float32,
                           precision=lax.Precision.HIGHEST)


_dense_pool = pl.pallas_call(
    _dense_pool_body,
    out_shape=jax.ShapeDtypeStruct((B, D), jnp.float32),
)


def kernel(x, edge_index, batch,
           W1_1, b1_1, gamma1, beta1, W1_2, b1_2,
           W4_1, b4_1, gamma4, beta4, W4_2, b4_2,
           W5_1, b5_1, gamma5, beta5, W5_2, b5_2):
    ppt = NCHUNK * CHUNK - E // NW
    pad_src = (jnp.arange(ppt, dtype=jnp.int32) * 41) % N
    pad_dst = PAD_DST + jnp.arange(ppt, dtype=jnp.int32)
    src = jnp.concatenate(
        [edge_index[0].astype(jnp.int32).reshape(NW, E // NW),
         jnp.broadcast_to(pad_src, (NW, ppt))], axis=1
    ).reshape(NW, NCHUNK, CHUNK)
    dst = jnp.concatenate(
        [edge_index[1].astype(jnp.int32).reshape(NW, E // NW),
         jnp.broadcast_to(pad_dst, (NW, ppt))], axis=1
    ).reshape(NW, NCHUNK, CHUNK)
    zeros = jnp.zeros((ZROWS, D), jnp.float32)
    batch32 = batch.astype(jnp.int32)
    starts = batch32[:-1].reshape(B, 1)
    ends = batch32[1:].reshape(B, 1)

    def layer(h, Wa, ba, g, bt, Wb, bb, last):
        agg = _sc_scatter(h, src, dst, zeros)
        ba2 = ba.reshape(1, D)
        g2 = g.reshape(1, D)
        bt2 = bt.reshape(1, D)
        bb2 = bb.reshape(1, D)
        if last:
            return _dense_pool(h, agg, Wa, ba2, g2, bt2, Wb, bb2,
                               starts, ends)
        return _dense(h, agg, Wa, ba2, g2, bt2, Wb, bb2)

    h = layer(x, W1_1, b1_1, gamma1, beta1, W1_2, b1_2, False)
    h = layer(h, W4_1, b4_1, gamma4, beta4, W4_2, b4_2, False)
    out = layer(h, W5_1, b5_1, gamma5, beta5, W5_2, b5_2, True)
    return out


# speedup vs baseline: 4.2974x; 1.0438x over previous
import functools

import jax
import jax.numpy as jnp
from jax import lax
from jax.experimental import pallas as pl
from jax.experimental.pallas import tpu as pltpu
from jax.experimental.pallas import tpu_sc as plsc

N = 10000
E = 320000
D = 128
B = 64

NC = 2
NS = 16
NW = NC * NS
CHUNK = 64
NCHUNK = 160
EPAD = NW * NCHUNK * CHUNK
NPAD = 10240
ZROWS = NPAD // NS
PAD_DST = N


PCH = NCHUNK // 4
NBUF = 4


def _sc_scatter_body(x_hbm, src_hbm, dst_hbm, zeros_hbm, out_hbm,
                     src_v, dst_v, gbuf0, gbuf1, gbuf2, gbuf3, agg,
                     sem0, sem1, sem2, sem3):
    c = lax.axis_index("c")
    s = lax.axis_index("s")
    wid = c * NS + s
    gbufs = (gbuf0, gbuf1, gbuf2, gbuf3)
    gsems = (sem0, sem1, sem2, sem3)

    def wait_gather(b):
        pltpu.make_async_copy(x_hbm.at[pl.ds(0, CHUNK)], gbufs[b],
                              gsems[b]).wait()

    pltpu.sync_copy(zeros_hbm, agg.at[pl.ds(s * ZROWS, ZROWS)])
    plsc.subcore_barrier()

    for p in range(NCHUNK // PCH):
        pltpu.sync_copy(src_hbm.at[wid, pl.ds(p * PCH, PCH)], src_v)
        pltpu.sync_copy(dst_hbm.at[wid, pl.ds(p * PCH, PCH)], dst_v)
        for b in range(NBUF):
            pltpu.async_copy(x_hbm.at[src_v.at[b]], gbufs[b], gsems[b])

        def body(jj, carry):
            for b in range(NBUF):
                j = jj * NBUF + b
                wait_gather(b)
                pltpu.sync_copy(gbufs[b], agg.at[dst_v.at[j]], add=True)
                pltpu.async_copy(x_hbm.at[src_v.at[j + NBUF]], gbufs[b],
                                 gsems[b])
            return carry

        lax.fori_loop(0, PCH // NBUF - 1, body, 0)
        for b in range(NBUF):
            j = PCH - NBUF + b
            wait_gather(b)
            pltpu.sync_copy(gbufs[b], agg.at[dst_v.at[j]], add=True)

    plsc.subcore_barrier()

    pltpu.sync_copy(agg.at[pl.ds(s * ZROWS, ZROWS)],
                    out_hbm.at[pl.ds(c * NPAD + s * ZROWS, ZROWS)])


_sc_scatter = pl.kernel(
    _sc_scatter_body,
    out_type=jax.ShapeDtypeStruct((NC * NPAD, D), jnp.float32),
    mesh=plsc.VectorSubcoreMesh(core_axis_name="c", subcore_axis_name="s"),
    scratch_types=[
        pltpu.VMEM((PCH, CHUNK), jnp.int32),
        pltpu.VMEM((PCH, CHUNK), jnp.int32),
        pltpu.VMEM((CHUNK, D), jnp.float32),
        pltpu.VMEM((CHUNK, D), jnp.float32),
        pltpu.VMEM((CHUNK, D), jnp.float32),
        pltpu.VMEM((CHUNK, D), jnp.float32),
        pltpu.VMEM_SHARED((NPAD, D), jnp.float32),
        pltpu.SemaphoreType.DMA,
        pltpu.SemaphoreType.DMA,
        pltpu.SemaphoreType.DMA,
        pltpu.SemaphoreType.DMA,
    ],
)


def _dense_body(x_ref, agg_ref, Wa_ref, ba_ref, g_ref, bt_ref,
                Wb_ref, bb_ref, out_ref):
    h = x_ref[...] + agg_ref[0:N] + agg_ref[NPAD:NPAD + N]
    h1 = jnp.dot(h, Wa_ref[...], preferred_element_type=jnp.float32) + ba_ref[...]
    m = jnp.mean(h1, axis=0, keepdims=True)
    v = jnp.mean((h1 - m) ** 2, axis=0, keepdims=True)
    hn = (h1 - m) * lax.rsqrt(v + 1e-5) * g_ref[...] + bt_ref[...]
    hn = jnp.maximum(hn, 0.0)
    h2 = jnp.dot(hn, Wb_ref[...], preferred_element_type=jnp.float32) + bb_ref[...]
    out_ref[...] = jnp.maximum(h2, 0.0)


_dense = pl.pallas_call(
    _dense_body,
    out_shape=jax.ShapeDtypeStruct((N, D), jnp.float32),
)


def _dense_pool_body(x_ref, agg_ref, Wa_ref, ba_ref, g_ref, bt_ref,
                     Wb_ref, bb_ref, starts_ref, ends_ref, out_ref):
    h = x_ref[...] + agg_ref[0:N] + agg_ref[NPAD:NPAD + N]
    h1 = jnp.dot(h, Wa_ref[...], preferred_element_type=jnp.float32) + ba_ref[...]
    m = jnp.mean(h1, axis=0, keepdims=True)
    v = jnp.mean((h1 - m) ** 2, axis=0, keepdims=True)
    hn = (h1 - m) * lax.rsqrt(v + 1e-5) * g_ref[...] + bt_ref[...]
    hn = jnp.maximum(hn, 0.0)
    h2 = jnp.dot(hn, Wb_ref[...], preferred_element_type=jnp.float32) + bb_ref[...]
    h2 = jnp.maximum(h2, 0.0)
    rows = lax.broadcasted_iota(jnp.int32, (B, N), 1)
    mask = (rows >= starts_ref[...]) & (rows < ends_ref[...])
    out_ref[...] = jnp.dot(mask.astype(jnp.float32), h2,
                           preferred_element_type=jnp.float32,
                           precision=lax.Precision.HIGHEST)


_dense_pool = pl.pallas_call(
    _dense_pool_body,
    out_shape=jax.ShapeDtypeStruct((B, D), jnp.float32),
)


def kernel(x, edge_index, batch,
           W1_1, b1_1, gamma1, beta1, W1_2, b1_2,
           W4_1, b4_1, gamma4, beta4, W4_2, b4_2,
           W5_1, b5_1, gamma5, beta5, W5_2, b5_2):
    ppt = NCHUNK * CHUNK - E // NW
    pad_src = (jnp.arange(ppt, dtype=jnp.int32) * 41) % N
    pad_dst = PAD_DST + jnp.arange(ppt, dtype=jnp.int32)
    src = jnp.concatenate(
        [edge_index[0].astype(jnp.int32).reshape(NW, E // NW),
         jnp.broadcast_to(pad_src, (NW, ppt))], axis=1
    ).reshape(NW, NCHUNK, CHUNK)
    dst = jnp.concatenate(
        [edge_index[1].astype(jnp.int32).reshape(NW, E // NW),
         jnp.broadcast_to(pad_dst, (NW, ppt))], axis=1
    ).reshape(NW, NCHUNK, CHUNK)
    zeros = jnp.zeros((ZROWS, D), jnp.float32)
    batch32 = batch.astype(jnp.int32)
    starts = batch32[:-1].reshape(B, 1)
    ends = batch32[1:].reshape(B, 1)

    def layer(h, Wa, ba, g, bt, Wb, bb, last):
        agg = _sc_scatter(h, src, dst, zeros)
        ba2 = ba.reshape(1, D)
        g2 = g.reshape(1, D)
        bt2 = bt.reshape(1, D)
        bb2 = bb.reshape(1, D)
        if last:
            return _dense_pool(h, agg, Wa, ba2, g2, bt2, Wb, bb2,
                               starts, ends)
        return _dense(h, agg, Wa, ba2, g2, bt2, Wb, bb2)

    h = layer(x, W1_1, b1_1, gamma1, beta1, W1_2, b1_2, False)
    h = layer(h, W4_1, b4_1, gamma4, beta4, W4_2, b4_2, False)
    out = layer(h, W5_1, b5_1, gamma5, beta5, W5_2, b5_2, True)
    return out
